# Initial kernel scaffold; baseline (speedup 1.0000x reference)
import jax
import jax.numpy as jnp
from jax.experimental import pallas as pl


def kernel(position_ids, maps):
    raise NotImplementedError("write your pallas kernel here")


# speedup vs baseline: 1.2087x; 1.2087x over previous
import functools

import jax
import jax.numpy as jnp
from jax import lax
from jax.experimental import pallas as pl
from jax.experimental.pallas import tpu as pltpu
from jax.experimental.pallas import tpu_sc as plsc

DIM = 64
B = 4096
NC, NS = 2, 16
NW = NC * NS
BPW = B // NW
CHUNK = 8
NCHUNK = BPW // CHUNK

_MESH = plsc.VectorSubcoreMesh(core_axis_name="c", subcore_axis_name="s")


@functools.partial(
    pl.kernel,
    mesh=_MESH,
    out_type=jax.ShapeDtypeStruct((B, DIM * DIM), jnp.float32),
    scratch_types=[
        pltpu.VMEM((NCHUNK, CHUNK), jnp.int32),
        pltpu.VMEM((CHUNK, DIM * DIM), jnp.float32),
        pltpu.VMEM((CHUNK, DIM * DIM), jnp.float32),
        pltpu.SemaphoreType.DMA,
        pltpu.SemaphoreType.DMA,
        pltpu.SemaphoreType.DMA,
        pltpu.SemaphoreType.DMA,
    ],
)
def _gather_rows(idx_hbm, maps_hbm, out_hbm, idx_v, buf0, buf1,
                 gsem0, gsem1, wsem0, wsem1):
    wid = lax.axis_index("s") * NC + lax.axis_index("c")
    base = wid * BPW

    pltpu.sync_copy(idx_hbm.at[wid], idx_v)

    bufs = (buf0, buf1)
    gsems = (gsem0, gsem1)
    wsems = (wsem0, wsem1)
    gathers = [None, None]
    writes = [None, None]

    gathers[0] = pltpu.async_copy(maps_hbm.at[idx_v.at[0]], bufs[0], gsems[0])

    for ci in range(NCHUNK):
        b = ci % 2
        nb = (ci + 1) % 2
        if ci + 1 < NCHUNK:
            if writes[nb] is not None:
                writes[nb].wait()
                writes[nb] = None
            gathers[nb] = pltpu.async_copy(
                maps_hbm.at[idx_v.at[ci + 1]], bufs[nb], gsems[nb])
        gathers[b].wait()
        writes[b] = pltpu.async_copy(
            bufs[b], out_hbm.at[pl.ds(base + ci * CHUNK, CHUNK)], wsems[b])

    writes[0].wait()
    writes[1].wait()


def kernel(position_ids, maps):
    idx = position_ids.reshape(NW, NCHUNK, CHUNK).astype(jnp.int32)
    maps2d = maps.reshape(maps.shape[0], DIM * DIM)
    out = _gather_rows(idx, maps2d)
    return out.reshape(1, 1, B, DIM, DIM)
